# baseline (speedup 1.0000x reference)
import functools

import jax
import jax.numpy as jnp
from jax import lax
from jax.experimental import pallas as pl
from jax.experimental.pallas import tpu as pltpu
from jax.experimental.pallas import tpu_sc as plsc

K = 16
BLK = 1024
CHUNK = 128
NCPB = BLK // CHUNK
EPS = 1e-12
NEG_INF = float("-inf")


def _simtopk_body(nblk, cap, q_ref, k_ref, sim_ref, cmax_ref, bids_ref,
                  qn_ref, bmax_ref):
    j = pl.program_id(0)
    qrows = q_ref.shape[0]

    @pl.when(j == 0)
    def _():
        q = q_ref[...]
        qn = jnp.sqrt(jnp.sum(q * q, axis=1, keepdims=True))
        qn_ref[...] = (q / jnp.maximum(qn, EPS)).astype(jnp.bfloat16)
        bmax_ref[...] = jnp.full(bmax_ref.shape, NEG_INF, jnp.float32)

    kblk = k_ref[...]
    knorm = jnp.sqrt(jnp.sum(kblk * kblk, axis=1, keepdims=True))
    kn = kblk / jnp.maximum(knorm, EPS)
    s = lax.dot_general(
        qn_ref[...], kn.astype(jnp.bfloat16),
        dimension_numbers=(((1,), (1,)), ((), ())),
        preferred_element_type=jnp.float32,
    )
    lane = lax.broadcasted_iota(jnp.int32, bmax_ref.shape, 1)

    def _tail(s3v):
        sim_ref[...] = s3v
        cm = jnp.max(s3v, axis=2)
        cmax_ref[0] = jnp.full((qrows, 128), NEG_INF, jnp.float32)
        cmax_ref[0, :, 0:NCPB] = cm
        bm = jnp.max(cm, axis=1, keepdims=True)
        bmax_ref[...] = jnp.where(lane == j, bm, bmax_ref[...])

    if cap % BLK:
        @pl.when(j == nblk - 1)
        def _():
            col = j * BLK + lax.broadcasted_iota(jnp.int32, s.shape, 1)
            _tail(jnp.where(col < cap, s, NEG_INF).reshape(qrows, NCPB, CHUNK))

        @pl.when(j != nblk - 1)
        def _():
            _tail(s.reshape(qrows, NCPB, CHUNK))
    else:
        _tail(s.reshape(qrows, NCPB, CHUNK))

    @pl.when(j == nblk - 1)
    def _():
        x = bmax_ref[...]
        big = jnp.int32(2147483647)
        for t in range(K):
            m = jnp.max(x, axis=1, keepdims=True)
            sel = jnp.min(jnp.where(x == m, lane, big), axis=1, keepdims=True)
            bids_ref[:, pl.ds(t, 1)] = sel
            x = jnp.where(lane == sel, NEG_INF, x)


def _sim_and_blocktopk(query, keys, cap):
    qrows, d = query.shape
    cpad = ((cap + BLK - 1) // BLK) * BLK
    nblk = cpad // BLK
    return pl.pallas_call(
        functools.partial(_simtopk_body, nblk, cap),
        grid=(nblk,),
        in_specs=[
            pl.BlockSpec((qrows, d), lambda j: (0, 0)),
            pl.BlockSpec((BLK, d), lambda j: (j, 0)),
        ],
        out_specs=[
            pl.BlockSpec((qrows, NCPB, CHUNK), lambda j: (0, j, 0)),
            pl.BlockSpec((1, qrows, 128), lambda j: (j, 0, 0)),
            pl.BlockSpec((qrows, K), lambda j: (0, 0)),
        ],
        out_shape=[
            jax.ShapeDtypeStruct((qrows, cpad // CHUNK, CHUNK), jnp.float32),
            jax.ShapeDtypeStruct((nblk, qrows, 128), jnp.float32),
            jax.ShapeDtypeStruct((qrows, K), jnp.int32),
        ],
        scratch_shapes=[
            pltpu.VMEM((qrows, d), jnp.bfloat16),
            pltpu.VMEM((qrows, 128), jnp.float32),
        ],
        compiler_params=pltpu.CompilerParams(
            dimension_semantics=("arbitrary",),
        ),
    )(query, keys)


def _select_body(x_ref, g_ref, val_ref, idx_ref):
    x = x_ref[...]
    g = g_ref[...]
    big = jnp.int32(2147483647)
    for t in range(K):
        m = jnp.max(x, axis=1, keepdims=True)
        sel = jnp.min(jnp.where(x == m, g, big), axis=1, keepdims=True)
        val_ref[:, pl.ds(t, 1)] = m
        idx_ref[:, pl.ds(t, 1)] = sel
        x = jnp.where(g == sel, NEG_INF, x)


def _select_topk(x, g):
    qrows = x.shape[0]
    return pl.pallas_call(
        _select_body,
        out_shape=[
            jax.ShapeDtypeStruct((qrows, K), jnp.float32),
            jax.ShapeDtypeStruct((qrows, K), jnp.int32),
        ],
    )(x, g)


def _sc_gather(table, idx2d):
    nrow_blocks = idx2d.shape[0]
    d = table.shape[1]
    nw = 32
    pwb = nrow_blocks // nw
    mesh = plsc.VectorSubcoreMesh(core_axis_name="c", subcore_axis_name="s")

    @functools.partial(
        pl.kernel,
        mesh=mesh,
        out_type=jax.ShapeDtypeStruct((nrow_blocks * 128, d), jnp.float32),
        scratch_types=[
            pltpu.VMEM((pwb, 128), jnp.int32),
            pltpu.VMEM((128, d), jnp.float32),
            pltpu.SemaphoreType.DMA,
        ],
    )
    def gk(table_hbm, idx_hbm, out_hbm, idx_v, rows_v, sem):
        cidx = lax.axis_index("c")
        sidx = lax.axis_index("s")
        wid = sidx * 2 + cidx
        blk0 = wid * pwb
        pltpu.sync_copy(idx_hbm.at[pl.ds(blk0, pwb)], idx_v)
        for jj in range(pwb):
            pltpu.async_copy(table_hbm.at[idx_v.at[jj]], rows_v, sem).wait()
            pltpu.sync_copy(rows_v, out_hbm.at[pl.ds((blk0 + jj) * 128, 128)])

    return gk(table, idx2d)


def _sc_gather_pack16(table, idx2d):
    nrow_blocks = idx2d.shape[0]
    nw = 32
    pwb = nrow_blocks // nw
    mesh = plsc.VectorSubcoreMesh(core_axis_name="c", subcore_axis_name="s")

    @functools.partial(
        pl.kernel,
        mesh=mesh,
        out_type=jax.ShapeDtypeStruct((nrow_blocks * 16, 128), jnp.float32),
        scratch_types=[
            pltpu.VMEM((pwb, 128), jnp.int32),
            pltpu.VMEM((128, 128), jnp.float32),
            pltpu.VMEM((16, 128), jnp.float32),
            pltpu.SemaphoreType.DMA,
        ],
    )
    def gk(table_hbm, idx_hbm, out_hbm, idx_v, rows_v, pack_v, sem):
        cidx = lax.axis_index("c")
        sidx = lax.axis_index("s")
        wid = sidx * 2 + cidx
        blk0 = wid * pwb
        pltpu.sync_copy(idx_hbm.at[pl.ds(blk0, pwb)], idx_v)
        for jj in range(pwb):
            pltpu.async_copy(table_hbm.at[idx_v.at[jj]], rows_v, sem).wait()
            for i in range(128):
                pack_v[i // 8, pl.ds((i % 8) * 16, 16)] = rows_v[i, 0:16]
            pltpu.sync_copy(pack_v, out_hbm.at[pl.ds((blk0 + jj) * 16, 16)])

    return gk(table, idx2d)


def kernel(query, keys, values, k):
    qrows, d = query.shape
    cap = keys.shape[0]
    dv = values.shape[1]

    cpad = ((cap + BLK - 1) // BLK) * BLK
    nblk = cpad // BLK
    nchunks = cpad // CHUNK

    sim3, cmax3, block_ids = _sim_and_blocktopk(query, keys, cap)

    g1_idx = (block_ids * qrows
              + jnp.arange(qrows, dtype=jnp.int32)[:, None]).reshape(-1)
    candmax = _sc_gather_pack16(cmax3.reshape(nblk * qrows, 128),
                                g1_idx.reshape(-1, 128))
    candmax = candmax.reshape(qrows, K * 16)
    gcid = (block_ids[:, :, None] * NCPB
            + (jnp.arange(16, dtype=jnp.int32) % NCPB)[None, None, :]
            ).reshape(qrows, K * 16)
    _, chunk_ids = _select_topk(candmax, gcid)

    g2_idx = (jnp.arange(qrows, dtype=jnp.int32)[:, None] * nchunks
              + chunk_ids).reshape(-1)
    cand = _sc_gather(sim3.reshape(qrows * nchunks, CHUNK),
                      g2_idx.reshape(-1, 128))
    cand = cand.reshape(qrows, K * CHUNK)
    gidx = (chunk_ids[:, :, None] * CHUNK
            + jnp.arange(CHUNK, dtype=jnp.int32)[None, None, :]
            ).reshape(qrows, K * CHUNK)
    scores, indices = _select_topk(cand, gidx)

    rows = _sc_gather(values, indices.reshape(-1, 128))
    retrieved = rows.reshape(qrows, K, dv)
    return retrieved, scores


# speedup vs baseline: 1.1252x; 1.1252x over previous
import functools

import jax
import jax.numpy as jnp
from jax import lax
from jax.experimental import pallas as pl
from jax.experimental.pallas import tpu as pltpu
from jax.experimental.pallas import tpu_sc as plsc

K = 16
BLK = 2048
CHUNK = 128
NCPB = BLK // CHUNK
EPS = 1e-12
NEG_INF = float("-inf")


def _simtopk_body(nblk, cap, q_ref, k_ref, sim_ref, cmax_ref, bids_ref,
                  qn_ref, bmax_ref):
    j = pl.program_id(0)
    qrows = q_ref.shape[0]

    @pl.when(j == 0)
    def _():
        q = q_ref[...]
        qn = jnp.sqrt(jnp.sum(q * q, axis=1, keepdims=True))
        qn_ref[...] = (q / jnp.maximum(qn, EPS)).astype(jnp.bfloat16)
        bmax_ref[...] = jnp.full(bmax_ref.shape, NEG_INF, jnp.float32)

    kblk = k_ref[...]
    knorm = jnp.sqrt(jnp.sum(kblk * kblk, axis=1, keepdims=True))
    kn = kblk / jnp.maximum(knorm, EPS)
    s = lax.dot_general(
        qn_ref[...], kn.astype(jnp.bfloat16),
        dimension_numbers=(((1,), (1,)), ((), ())),
        preferred_element_type=jnp.float32,
    )
    lane = lax.broadcasted_iota(jnp.int32, bmax_ref.shape, 1)

    def _tail(s3v):
        sim_ref[...] = s3v
        cm = jnp.max(s3v, axis=2)
        cmax_ref[0] = jnp.full((qrows, 128), NEG_INF, jnp.float32)
        cmax_ref[0, :, 0:NCPB] = cm
        bm = jnp.max(cm, axis=1, keepdims=True)
        bmax_ref[...] = jnp.where(lane == j, bm, bmax_ref[...])

    if cap % BLK:
        @pl.when(j == nblk - 1)
        def _():
            col = j * BLK + lax.broadcasted_iota(jnp.int32, s.shape, 1)
            _tail(jnp.where(col < cap, s, NEG_INF).reshape(qrows, NCPB, CHUNK))

        @pl.when(j != nblk - 1)
        def _():
            _tail(s.reshape(qrows, NCPB, CHUNK))
    else:
        _tail(s.reshape(qrows, NCPB, CHUNK))

    @pl.when(j == nblk - 1)
    def _():
        x = bmax_ref[...]
        big = jnp.int32(2147483647)
        for t in range(K):
            m = jnp.max(x, axis=1, keepdims=True)
            sel = jnp.min(jnp.where(x == m, lane, big), axis=1, keepdims=True)
            bids_ref[:, pl.ds(t, 1)] = sel
            x = jnp.where(lane == sel, NEG_INF, x)


def _sim_and_blocktopk(query, keys, cap):
    qrows, d = query.shape
    cpad = ((cap + BLK - 1) // BLK) * BLK
    nblk = cpad // BLK
    return pl.pallas_call(
        functools.partial(_simtopk_body, nblk, cap),
        grid=(nblk,),
        in_specs=[
            pl.BlockSpec((qrows, d), lambda j: (0, 0)),
            pl.BlockSpec((BLK, d), lambda j: (j, 0)),
        ],
        out_specs=[
            pl.BlockSpec((qrows, NCPB, CHUNK), lambda j: (0, j, 0)),
            pl.BlockSpec((1, qrows, 128), lambda j: (j, 0, 0)),
            pl.BlockSpec((qrows, K), lambda j: (0, 0)),
        ],
        out_shape=[
            jax.ShapeDtypeStruct((qrows, cpad // CHUNK, CHUNK), jnp.float32),
            jax.ShapeDtypeStruct((nblk, qrows, 128), jnp.float32),
            jax.ShapeDtypeStruct((qrows, K), jnp.int32),
        ],
        scratch_shapes=[
            pltpu.VMEM((qrows, d), jnp.bfloat16),
            pltpu.VMEM((qrows, 128), jnp.float32),
        ],
        compiler_params=pltpu.CompilerParams(
            dimension_semantics=("arbitrary",),
        ),
    )(query, keys)


def _select_body(x_ref, g_ref, val_ref, idx_ref):
    x = x_ref[...]
    g = g_ref[...]
    big = jnp.int32(2147483647)
    for t in range(K):
        m = jnp.max(x, axis=1, keepdims=True)
        sel = jnp.min(jnp.where(x == m, g, big), axis=1, keepdims=True)
        val_ref[:, pl.ds(t, 1)] = m
        idx_ref[:, pl.ds(t, 1)] = sel
        x = jnp.where(g == sel, NEG_INF, x)


def _select_topk(x, g):
    qrows = x.shape[0]
    return pl.pallas_call(
        _select_body,
        out_shape=[
            jax.ShapeDtypeStruct((qrows, K), jnp.float32),
            jax.ShapeDtypeStruct((qrows, K), jnp.int32),
        ],
    )(x, g)


def _sc_gather(table, idx2d):
    nrow_blocks = idx2d.shape[0]
    d = table.shape[1]
    nw = 32
    pwb = nrow_blocks // nw
    mesh = plsc.VectorSubcoreMesh(core_axis_name="c", subcore_axis_name="s")

    @functools.partial(
        pl.kernel,
        mesh=mesh,
        out_type=jax.ShapeDtypeStruct((nrow_blocks * 128, d), jnp.float32),
        scratch_types=[
            pltpu.VMEM((pwb, 128), jnp.int32),
            pltpu.VMEM((128, d), jnp.float32),
            pltpu.SemaphoreType.DMA,
        ],
    )
    def gk(table_hbm, idx_hbm, out_hbm, idx_v, rows_v, sem):
        cidx = lax.axis_index("c")
        sidx = lax.axis_index("s")
        wid = sidx * 2 + cidx
        blk0 = wid * pwb
        pltpu.sync_copy(idx_hbm.at[pl.ds(blk0, pwb)], idx_v)
        for jj in range(pwb):
            pltpu.async_copy(table_hbm.at[idx_v.at[jj]], rows_v, sem).wait()
            pltpu.sync_copy(rows_v, out_hbm.at[pl.ds((blk0 + jj) * 128, 128)])

    return gk(table, idx2d)


def _sc_gather_pack16(table, idx2d):
    nrow_blocks = idx2d.shape[0]
    nw = 32
    pwb = nrow_blocks // nw
    mesh = plsc.VectorSubcoreMesh(core_axis_name="c", subcore_axis_name="s")

    @functools.partial(
        pl.kernel,
        mesh=mesh,
        out_type=jax.ShapeDtypeStruct((nrow_blocks * 16, 128), jnp.float32),
        scratch_types=[
            pltpu.VMEM((pwb, 128), jnp.int32),
            pltpu.VMEM((128, 128), jnp.float32),
            pltpu.VMEM((16, 128), jnp.float32),
            pltpu.SemaphoreType.DMA,
        ],
    )
    def gk(table_hbm, idx_hbm, out_hbm, idx_v, rows_v, pack_v, sem):
        cidx = lax.axis_index("c")
        sidx = lax.axis_index("s")
        wid = sidx * 2 + cidx
        blk0 = wid * pwb
        pltpu.sync_copy(idx_hbm.at[pl.ds(blk0, pwb)], idx_v)
        for jj in range(pwb):
            pltpu.async_copy(table_hbm.at[idx_v.at[jj]], rows_v, sem).wait()
            for i in range(128):
                pack_v[i // 8, pl.ds((i % 8) * 16, 16)] = rows_v[i, 0:16]
            pltpu.sync_copy(pack_v, out_hbm.at[pl.ds((blk0 + jj) * 16, 16)])

    return gk(table, idx2d)


def kernel(query, keys, values, k):
    qrows, d = query.shape
    cap = keys.shape[0]
    dv = values.shape[1]

    cpad = ((cap + BLK - 1) // BLK) * BLK
    nblk = cpad // BLK
    nchunks = cpad // CHUNK

    sim3, cmax3, block_ids = _sim_and_blocktopk(query, keys, cap)

    g1_idx = (block_ids * qrows
              + jnp.arange(qrows, dtype=jnp.int32)[:, None]).reshape(-1)
    candmax = _sc_gather_pack16(cmax3.reshape(nblk * qrows, 128),
                                g1_idx.reshape(-1, 128))
    candmax = candmax.reshape(qrows, K * 16)
    gcid = (block_ids[:, :, None] * NCPB
            + (jnp.arange(16, dtype=jnp.int32) % NCPB)[None, None, :]
            ).reshape(qrows, K * 16)
    _, chunk_ids = _select_topk(candmax, gcid)

    g2_idx = (jnp.arange(qrows, dtype=jnp.int32)[:, None] * nchunks
              + chunk_ids).reshape(-1)
    cand = _sc_gather(sim3.reshape(qrows * nchunks, CHUNK),
                      g2_idx.reshape(-1, 128))
    cand = cand.reshape(qrows, K * CHUNK)
    gidx = (chunk_ids[:, :, None] * CHUNK
            + jnp.arange(CHUNK, dtype=jnp.int32)[None, None, :]
            ).reshape(qrows, K * CHUNK)
    scores, indices = _select_topk(cand, gidx)

    rows = _sc_gather(values, indices.reshape(-1, 128))
    retrieved = rows.reshape(qrows, K, dv)
    return retrieved, scores
